# Initial kernel scaffold; baseline (speedup 1.0000x reference)
import jax
import jax.numpy as jnp
from jax.experimental import pallas as pl


def kernel(xyz_pc, params):
    raise NotImplementedError("write your pallas kernel here")


# speedup vs baseline: 1.8653x; 1.8653x over previous
import functools

import jax
import jax.numpy as jnp
import numpy as np
from jax.experimental import pallas as pl


def _mlp_body(nlayers, relu_flags, pool_k, bm, x_ref, *refs):
    out_ref = refs[-1]
    h = x_ref[...]
    for li in range(nlayers):
        w = refs[2 * li][...]
        b = refs[2 * li + 1][...]
        h = jnp.dot(h, w, preferred_element_type=jnp.float32) + b
        if relu_flags[li]:
            h = jnp.maximum(h, 0.0)
    if pool_k is None:
        out_ref[...] = h
    else:
        g = bm // pool_k
        rows = [
            jnp.max(h[i * pool_k:(i + 1) * pool_k, :], axis=0, keepdims=True)
            for i in range(g)
        ]
        out_ref[...] = jnp.concatenate(rows, axis=0) if g > 1 else rows[0]


def _mlp_pallas(x2d, layers, pool_k=None):
    m, cin = x2d.shape
    if pool_k is None:
        bm = m if m < 512 else 512
    else:
        bm = min(16 * pool_k, m)
        assert bm % pool_k == 0
    grid = m // bm
    in_specs = [pl.BlockSpec((bm, cin), lambda i: (i, 0))]
    operands = [x2d]
    relu_flags = []
    for (wt, b, relu) in layers:
        in_specs.append(pl.BlockSpec(wt.shape, lambda i: (0, 0)))
        in_specs.append(pl.BlockSpec(b.shape, lambda i: (0, 0)))
        operands.extend([wt, b])
        relu_flags.append(relu)
    cout = layers[-1][0].shape[1]
    if pool_k is None:
        out_shape = jax.ShapeDtypeStruct((m, cout), jnp.float32)
        out_spec = pl.BlockSpec((bm, cout), lambda i: (i, 0))
    else:
        out_shape = jax.ShapeDtypeStruct((m // pool_k, cout), jnp.float32)
        out_spec = pl.BlockSpec((bm // pool_k, cout), lambda i: (i, 0))
    body = functools.partial(_mlp_body, len(layers), tuple(relu_flags), pool_k, bm)
    return pl.pallas_call(
        body,
        grid=(grid,),
        in_specs=in_specs,
        out_specs=out_spec,
        out_shape=out_shape,
    )(*operands)


def _fold_layers(layers, last_plain=None):
    out = []
    inv = np.float32(1.0 / np.sqrt(1.0 + 1e-5))
    for (w, b, g, be) in layers:
        s = g * inv
        wt = (w * s[:, None]).T
        bb = (b * s + be)[None, :]
        out.append((wt, bb, True))
    if last_plain is not None:
        w2, b2 = last_plain
        out.append((w2.T, b2[None, :], False))
    return out


def _fps(xyz, npoint):
    n = xyz.shape[0]

    def step(state, _):
        distance, farthest = state
        centroid = xyz[farthest][None, :]
        dist = jnp.sum((xyz - centroid) ** 2, -1)
        distance = jnp.minimum(distance, dist)
        new_far = jnp.argmax(distance, -1).astype(jnp.int32)
        return (distance, new_far), farthest

    init = (jnp.full((n,), 1e10, dtype=xyz.dtype), jnp.zeros((), jnp.int32))
    _, cent = jax.lax.scan(step, init, None, length=npoint)
    return cent


def _query_ball(radius, nsample, xyz, new_xyz):
    n = xyz.shape[0]
    s = new_xyz.shape[0]
    sqr = (
        jnp.sum(new_xyz ** 2, -1)[:, None]
        + jnp.sum(xyz ** 2, -1)[None, :]
        - 2.0 * jnp.dot(new_xyz, xyz.T)
    )
    gi = jnp.broadcast_to(jnp.arange(n, dtype=jnp.int32)[None, :], (s, n))
    gi = jnp.where(sqr > radius ** 2, n, gi)
    gi = jnp.sort(gi, axis=-1)[:, :nsample]
    first = jnp.broadcast_to(gi[:, :1], gi.shape)
    return jnp.where(gi == n, first, gi)


def _sa_msg(xyz, points, npoint, radius_list, nsample_list, scale_params):
    fps_idx = _fps(xyz, npoint)
    new_xyz = xyz[fps_idx]
    outs = []
    for i, radius in enumerate(radius_list):
        k = nsample_list[i]
        gi = _query_ball(radius, k, xyz, new_xyz)
        grouped_xyz = xyz[gi] - new_xyz[:, None, :]
        grouped = jnp.concatenate([points[gi], grouped_xyz], axis=-1)
        s, _, c = grouped.shape
        feat = grouped.reshape(s * k, c)
        layers = _fold_layers(scale_params[i])
        outs.append(_mlp_pallas(feat, layers, pool_k=k))
    return new_xyz, jnp.concatenate(outs, axis=-1)


def _fp(xyz1, xyz2, points1, points2, layers):
    n = xyz1.shape[0]
    s = xyz2.shape[0]
    if s == 1:
        interp = jnp.broadcast_to(points2, (n, points2.shape[-1]))
    else:
        dists = (
            jnp.sum(xyz1 ** 2, -1)[:, None]
            + jnp.sum(xyz2 ** 2, -1)[None, :]
            - 2.0 * jnp.dot(xyz1, xyz2.T)
        )
        neg, idx = jax.lax.top_k(-dists, 3)
        d = -neg
        recip = 1.0 / (d + 1e-8)
        w = recip / jnp.sum(recip, -1, keepdims=True)
        interp = jnp.sum(points2[idx] * w[..., None], axis=1)
    x = jnp.concatenate([points1, interp], axis=-1)
    return _mlp_pallas(x, _fold_layers(layers))


def _head(x, p):
    layers = _fold_layers([p['l1']], last_plain=p['l2'])
    return _mlp_pallas(x, layers)


def _bin_vals():
    bb = np.array([0, 0.00794435329, 0.0158887021, 0.0238330509, 0.0317773996,
                   0.0397217484, 0.0476660972, 0.055610446, 0.0635547948,
                   0.0714991435, 0.08], dtype=np.float32)
    bv = (bb[:-1] + bb[1:]) / 2.0
    bv = np.minimum(bv, np.float32(0.08 - 0.005))
    return jnp.asarray(bv)


def _normalize(x):
    nrm = jnp.sqrt(jnp.sum(x * x, axis=-1, keepdims=True))
    return x / jnp.maximum(nrm, 1e-12)


def kernel(xyz_pc, params):
    xyz0 = xyz_pc[0]
    p = params

    l1_xyz, l1_points = _sa_msg(xyz0, xyz0, 2048, [0.02, 0.04, 0.08],
                                [32, 64, 128], p['sa1'])
    l2_xyz, l2_points = _sa_msg(l1_xyz, l1_points, 512, [0.04, 0.08, 0.16],
                                [64, 64, 128], p['sa2'])
    l3_xyz, l3_points = _sa_msg(l2_xyz, l2_points, 128, [0.08, 0.16, 0.32],
                                [64, 64, 128], p['sa3'])

    x4 = jnp.concatenate([l3_xyz, l3_points], axis=-1)
    l4_points = _mlp_pallas(x4, _fold_layers(p['sa4']), pool_k=x4.shape[0])
    l4_xyz = jnp.zeros((1, 3), xyz0.dtype)

    l3_points = _fp(l3_xyz, l4_xyz, l3_points, l4_points, p['fp3'])
    l2_points = _fp(l2_xyz, l3_xyz, l2_points, l3_points, p['fp2'])
    l1_points = _fp(l1_xyz, l2_xyz, l1_points, l2_points, p['fp1'])

    pred_points = l1_xyz
    grasp_dir = _normalize(_head(l1_points, p['dir']))
    approach = _head(l1_points, p['app'])
    dot = jnp.sum(approach * grasp_dir, axis=-1, keepdims=True)
    approach = _normalize(approach - dot * grasp_dir)
    width_logits = _head(l1_points, p['off'])
    seg = _head(l1_points, p['seg'])

    bin_vals = _bin_vals()
    width_idx = jnp.argmax(width_logits, axis=-1)
    grasp_width = bin_vals[width_idx][..., None]
    gripper_depth = 0.1034
    grasp_r = jnp.stack([grasp_dir, jnp.cross(approach, grasp_dir), approach],
                        axis=2)
    grasp_t = (pred_points + grasp_width / 2.0 * grasp_dir
               - gripper_depth * approach)[..., None]
    np_ = approach.shape[0]
    homog = jnp.concatenate([jnp.zeros((np_, 1, 3), jnp.float32),
                             jnp.ones((np_, 1, 1), jnp.float32)], axis=2)
    pred_grasps = jnp.concatenate(
        [jnp.concatenate([grasp_r, grasp_t], axis=2), homog], axis=1)

    pred_scores = jax.nn.sigmoid(seg)
    return (pred_grasps[None], pred_scores[None], pred_points[None])


# speedup vs baseline: 2.6769x; 1.4351x over previous
import functools

import jax
import jax.numpy as jnp
import numpy as np
from jax.experimental import pallas as pl
from jax.experimental.pallas import tpu as pltpu


def _mlp_body(nlayers, relu_flags, pool_k, bm, x_ref, *refs):
    out_ref = refs[-1]
    h = x_ref[...]
    for li in range(nlayers):
        w = refs[2 * li][...]
        b = refs[2 * li + 1][...]
        h = jnp.dot(h, w, preferred_element_type=jnp.float32) + b
        if relu_flags[li]:
            h = jnp.maximum(h, 0.0)
    if pool_k is None:
        out_ref[...] = h
    else:
        g = bm // pool_k
        rows = [
            jnp.max(h[i * pool_k:(i + 1) * pool_k, :], axis=0, keepdims=True)
            for i in range(g)
        ]
        out_ref[...] = jnp.concatenate(rows, axis=0) if g > 1 else rows[0]


def _mlp_pallas(x2d, layers, pool_k=None):
    m, cin = x2d.shape
    if pool_k is None:
        bm = m if m < 512 else 512
    else:
        bm = min(16 * pool_k, m)
        assert bm % pool_k == 0
    grid = m // bm
    in_specs = [pl.BlockSpec((bm, cin), lambda i: (i, 0))]
    operands = [x2d]
    relu_flags = []
    for (wt, b, relu) in layers:
        in_specs.append(pl.BlockSpec(wt.shape, lambda i: (0, 0)))
        in_specs.append(pl.BlockSpec(b.shape, lambda i: (0, 0)))
        operands.extend([wt, b])
        relu_flags.append(relu)
    cout = layers[-1][0].shape[1]
    if pool_k is None:
        out_shape = jax.ShapeDtypeStruct((m, cout), jnp.float32)
        out_spec = pl.BlockSpec((bm, cout), lambda i: (i, 0))
    else:
        out_shape = jax.ShapeDtypeStruct((m // pool_k, cout), jnp.float32)
        out_spec = pl.BlockSpec((bm // pool_k, cout), lambda i: (i, 0))
    body = functools.partial(_mlp_body, len(layers), tuple(relu_flags), pool_k, bm)
    return pl.pallas_call(
        body,
        grid=(grid,),
        in_specs=in_specs,
        out_specs=out_spec,
        out_shape=out_shape,
    )(*operands)


def _fold_layers(layers, last_plain=None):
    out = []
    inv = np.float32(1.0 / np.sqrt(1.0 + 1e-5))
    for (w, b, g, be) in layers:
        s = g * inv
        wt = (w * s[:, None]).T
        bb = (b * s + be)[None, :]
        out.append((wt, bb, True))
    if last_plain is not None:
        w2, b2 = last_plain
        out.append((w2.T, b2[None, :], False))
    return out


def _fps_body(npoint, n, xyz_ref, out_ref, dist_ref):
    rows = n // 128
    xs = xyz_ref[0]
    ys = xyz_ref[1]
    zs = xyz_ref[2]
    iota2d = (jax.lax.broadcasted_iota(jnp.int32, (rows, 128), 0) * 128
              + jax.lax.broadcasted_iota(jnp.int32, (rows, 128), 1))
    dist_ref[...] = jnp.full((rows, 128), 1e10, jnp.float32)

    def step(t, far):
        out_ref[t] = far
        m = iota2d == far
        cx = jnp.sum(jnp.where(m, xs, 0.0))
        cy = jnp.sum(jnp.where(m, ys, 0.0))
        cz = jnp.sum(jnp.where(m, zs, 0.0))
        dx = xs - cx
        dy = ys - cy
        dz = zs - cz
        d = (dx * dx + dz * dz) + dy * dy
        dist = jnp.minimum(dist_ref[...], d)
        dist_ref[...] = dist
        mx = jnp.max(dist)
        return jnp.min(jnp.where(dist == mx, iota2d, n)).astype(jnp.int32)

    jax.lax.fori_loop(0, npoint, step, jnp.zeros((), jnp.int32))


def _fps(xyz, npoint):
    n = xyz.shape[0]
    planes = xyz.T.reshape(3, n // 128, 128)
    return pl.pallas_call(
        functools.partial(_fps_body, npoint, n),
        in_specs=[pl.BlockSpec(planes.shape, lambda: (0, 0, 0))],
        out_specs=pl.BlockSpec(memory_space=pltpu.SMEM),
        out_shape=jax.ShapeDtypeStruct((npoint,), jnp.int32),
        scratch_shapes=[pltpu.VMEM((n // 128, 128), jnp.float32)],
    )(planes)


def _query_ball(radius, nsample, xyz, new_xyz):
    n = xyz.shape[0]
    s = new_xyz.shape[0]
    sqr = (
        jnp.sum(new_xyz ** 2, -1)[:, None]
        + jnp.sum(xyz ** 2, -1)[None, :]
        - 2.0 * jnp.dot(new_xyz, xyz.T)
    )
    gi = jnp.broadcast_to(jnp.arange(n, dtype=jnp.int32)[None, :], (s, n))
    gi = jnp.where(sqr > radius ** 2, n, gi)
    gi = jnp.sort(gi, axis=-1)[:, :nsample]
    first = jnp.broadcast_to(gi[:, :1], gi.shape)
    return jnp.where(gi == n, first, gi)


def _sa_msg(xyz, points, npoint, radius_list, nsample_list, scale_params):
    fps_idx = _fps(xyz, npoint)
    new_xyz = xyz[fps_idx]
    outs = []
    for i, radius in enumerate(radius_list):
        k = nsample_list[i]
        gi = _query_ball(radius, k, xyz, new_xyz)
        grouped_xyz = xyz[gi] - new_xyz[:, None, :]
        grouped = jnp.concatenate([points[gi], grouped_xyz], axis=-1)
        s, _, c = grouped.shape
        feat = grouped.reshape(s * k, c)
        layers = _fold_layers(scale_params[i])
        outs.append(_mlp_pallas(feat, layers, pool_k=k))
    return new_xyz, jnp.concatenate(outs, axis=-1)


def _fp(xyz1, xyz2, points1, points2, layers):
    n = xyz1.shape[0]
    s = xyz2.shape[0]
    if s == 1:
        interp = jnp.broadcast_to(points2, (n, points2.shape[-1]))
    else:
        dists = (
            jnp.sum(xyz1 ** 2, -1)[:, None]
            + jnp.sum(xyz2 ** 2, -1)[None, :]
            - 2.0 * jnp.dot(xyz1, xyz2.T)
        )
        neg, idx = jax.lax.top_k(-dists, 3)
        d = -neg
        recip = 1.0 / (d + 1e-8)
        w = recip / jnp.sum(recip, -1, keepdims=True)
        interp = jnp.sum(points2[idx] * w[..., None], axis=1)
    x = jnp.concatenate([points1, interp], axis=-1)
    return _mlp_pallas(x, _fold_layers(layers))


def _head(x, p):
    layers = _fold_layers([p['l1']], last_plain=p['l2'])
    return _mlp_pallas(x, layers)


def _bin_vals():
    bb = np.array([0, 0.00794435329, 0.0158887021, 0.0238330509, 0.0317773996,
                   0.0397217484, 0.0476660972, 0.055610446, 0.0635547948,
                   0.0714991435, 0.08], dtype=np.float32)
    bv = (bb[:-1] + bb[1:]) / 2.0
    bv = np.minimum(bv, np.float32(0.08 - 0.005))
    return jnp.asarray(bv)


def _normalize(x):
    nrm = jnp.sqrt(jnp.sum(x * x, axis=-1, keepdims=True))
    return x / jnp.maximum(nrm, 1e-12)


def kernel(xyz_pc, params):
    xyz0 = xyz_pc[0]
    p = params

    l1_xyz, l1_points = _sa_msg(xyz0, xyz0, 2048, [0.02, 0.04, 0.08],
                                [32, 64, 128], p['sa1'])
    l2_xyz, l2_points = _sa_msg(l1_xyz, l1_points, 512, [0.04, 0.08, 0.16],
                                [64, 64, 128], p['sa2'])
    l3_xyz, l3_points = _sa_msg(l2_xyz, l2_points, 128, [0.08, 0.16, 0.32],
                                [64, 64, 128], p['sa3'])

    x4 = jnp.concatenate([l3_xyz, l3_points], axis=-1)
    l4_points = _mlp_pallas(x4, _fold_layers(p['sa4']), pool_k=x4.shape[0])
    l4_xyz = jnp.zeros((1, 3), xyz0.dtype)

    l3_points = _fp(l3_xyz, l4_xyz, l3_points, l4_points, p['fp3'])
    l2_points = _fp(l2_xyz, l3_xyz, l2_points, l3_points, p['fp2'])
    l1_points = _fp(l1_xyz, l2_xyz, l1_points, l2_points, p['fp1'])

    pred_points = l1_xyz
    grasp_dir = _normalize(_head(l1_points, p['dir']))
    approach = _head(l1_points, p['app'])
    dot = jnp.sum(approach * grasp_dir, axis=-1, keepdims=True)
    approach = _normalize(approach - dot * grasp_dir)
    width_logits = _head(l1_points, p['off'])
    seg = _head(l1_points, p['seg'])

    bin_vals = _bin_vals()
    width_idx = jnp.argmax(width_logits, axis=-1)
    grasp_width = bin_vals[width_idx][..., None]
    gripper_depth = 0.1034
    grasp_r = jnp.stack([grasp_dir, jnp.cross(approach, grasp_dir), approach],
                        axis=2)
    grasp_t = (pred_points + grasp_width / 2.0 * grasp_dir
               - gripper_depth * approach)[..., None]
    np_ = approach.shape[0]
    homog = jnp.concatenate([jnp.zeros((np_, 1, 3), jnp.float32),
                             jnp.ones((np_, 1, 1), jnp.float32)], axis=2)
    pred_grasps = jnp.concatenate(
        [jnp.concatenate([grasp_r, grasp_t], axis=2), homog], axis=1)

    pred_scores = jax.nn.sigmoid(seg)
    return (pred_grasps[None], pred_scores[None], pred_points[None])


# speedup vs baseline: 2.7522x; 1.0281x over previous
import functools

import jax
import jax.numpy as jnp
import numpy as np
from jax.experimental import pallas as pl
from jax.experimental.pallas import tpu as pltpu
from jax.experimental.pallas import tpu_sc as plsc


def _mlp_body(nlayers, relu_flags, pool_k, bm, x_ref, *refs):
    out_ref = refs[-1]
    h = x_ref[...]
    for li in range(nlayers):
        w = refs[2 * li][...]
        b = refs[2 * li + 1][...]
        h = jnp.dot(h, w, preferred_element_type=jnp.float32) + b
        if relu_flags[li]:
            h = jnp.maximum(h, 0.0)
    if pool_k is None:
        out_ref[...] = h
    else:
        g = bm // pool_k
        rows = [
            jnp.max(h[i * pool_k:(i + 1) * pool_k, :], axis=0, keepdims=True)
            for i in range(g)
        ]
        out_ref[...] = jnp.concatenate(rows, axis=0) if g > 1 else rows[0]


def _mlp_pallas(x2d, layers, pool_k=None):
    m, cin = x2d.shape
    if pool_k is None:
        bm = m if m < 512 else 512
    else:
        bm = min(16 * pool_k, m)
        assert bm % pool_k == 0
    grid = m // bm
    in_specs = [pl.BlockSpec((bm, cin), lambda i: (i, 0))]
    operands = [x2d]
    relu_flags = []
    for (wt, b, relu) in layers:
        in_specs.append(pl.BlockSpec(wt.shape, lambda i: (0, 0)))
        in_specs.append(pl.BlockSpec(b.shape, lambda i: (0, 0)))
        operands.extend([wt, b])
        relu_flags.append(relu)
    cout = layers[-1][0].shape[1]
    if pool_k is None:
        out_shape = jax.ShapeDtypeStruct((m, cout), jnp.float32)
        out_spec = pl.BlockSpec((bm, cout), lambda i: (i, 0))
    else:
        out_shape = jax.ShapeDtypeStruct((m // pool_k, cout), jnp.float32)
        out_spec = pl.BlockSpec((bm // pool_k, cout), lambda i: (i, 0))
    body = functools.partial(_mlp_body, len(layers), tuple(relu_flags), pool_k, bm)
    return pl.pallas_call(
        body,
        grid=(grid,),
        in_specs=in_specs,
        out_specs=out_spec,
        out_shape=out_shape,
    )(*operands)


def _fold_layers(layers, last_plain=None):
    out = []
    inv = np.float32(1.0 / np.sqrt(1.0 + 1e-5))
    for (w, b, g, be) in layers:
        s = g * inv
        wt = (w * s[:, None]).T
        bb = (b * s + be)[None, :]
        out.append((wt, bb, True))
    if last_plain is not None:
        w2, b2 = last_plain
        out.append((w2.T, b2[None, :], False))
    return out


def _fps_body(npoint, n, xyz_ref, out_ref, dist_ref):
    rows = n // 128
    xs = xyz_ref[0]
    ys = xyz_ref[1]
    zs = xyz_ref[2]
    iota2d = (jax.lax.broadcasted_iota(jnp.int32, (rows, 128), 0) * 128
              + jax.lax.broadcasted_iota(jnp.int32, (rows, 128), 1))
    dist_ref[...] = jnp.full((rows, 128), 1e10, jnp.float32)

    def step(t, far):
        out_ref[t] = far
        m = iota2d == far
        cx = jnp.sum(jnp.where(m, xs, 0.0))
        cy = jnp.sum(jnp.where(m, ys, 0.0))
        cz = jnp.sum(jnp.where(m, zs, 0.0))
        dx = xs - cx
        dy = ys - cy
        dz = zs - cz
        d = (dx * dx + dz * dz) + dy * dy
        dist = jnp.minimum(dist_ref[...], d)
        dist_ref[...] = dist
        mx = jnp.max(dist)
        return jnp.min(jnp.where(dist == mx, iota2d, n)).astype(jnp.int32)

    jax.lax.fori_loop(0, npoint, step, jnp.zeros((), jnp.int32))


def _fps(xyz, npoint):
    n = xyz.shape[0]
    planes = xyz.T.reshape(3, n // 128, 128)
    return pl.pallas_call(
        functools.partial(_fps_body, npoint, n),
        in_specs=[pl.BlockSpec(planes.shape, lambda: (0, 0, 0))],
        out_specs=pl.BlockSpec(memory_space=pltpu.SMEM),
        out_shape=jax.ShapeDtypeStruct((npoint,), jnp.int32),
        scratch_shapes=[pltpu.VMEM((n // 128, 128), jnp.float32)],
    )(planes)


def _bq_body(n, s_blk, radii2, ks, sqr_ref, tri_ref,
             o1_ref, o2_ref, o3_ref, rank_ref):
    sqr = sqr_ref[...]
    nc = n // 512 if n >= 512 else 1
    cw = n // nc
    out_refs = (o1_ref, o2_ref, o3_ref)
    for si in range(3):
        r2, k = radii2[si], ks[si]
        o_ref = out_refs[si]
        mf = jnp.where(sqr <= r2, 1.0, 0.0)
        run = jnp.zeros((s_blk, 1), jnp.float32)
        for c in range(nc):
            sl = pl.ds(c * cw, cw)
            rc = jnp.dot(mf[:, c * cw:(c + 1) * cw], tri_ref[0:cw, 0:cw],
                         preferred_element_type=jnp.float32) + run
            run = rc[:, cw - 1:cw]
            rank_ref[:, sl] = rc
        rank = rank_ref[...]
        kio = jax.lax.broadcasted_iota(jnp.int32, (s_blk, k), 1)

        def col(kk, acc):
            ck = jnp.sum(jnp.where(rank <= kk.astype(jnp.float32), 1.0, 0.0),
                         axis=1, keepdims=True)
            return acc + jnp.where(kio == kk, ck.astype(jnp.int32), 0)

        idx = jax.lax.fori_loop(0, k, col, jnp.zeros((s_blk, k), jnp.int32))
        idx0 = idx[:, 0:1]
        o_ref[...] = jnp.where(kio < run.astype(jnp.int32), idx,
                               jnp.broadcast_to(idx0, (s_blk, k)))


def _ball_query_tc(radius_list, nsample_list, xyz, new_xyz):
    n = xyz.shape[0]
    s = new_xyz.shape[0]
    ks = tuple(nsample_list)
    radii2 = tuple(np.float32(r ** 2) for r in radius_list)
    src_b = new_xyz[None]
    dst_b = xyz[None]
    sqr = (jnp.sum(src_b ** 2, -1)[:, :, None]
           + jnp.sum(dst_b ** 2, -1)[:, None, :]
           - 2.0 * jnp.einsum('bnc,bmc->bnm', src_b, dst_b))[0]
    cw = min(n, 512)
    tri = jnp.asarray(np.triu(np.ones((cw, cw), np.float32)))
    s_blk = 8
    grid = s // s_blk
    outs = pl.pallas_call(
        functools.partial(_bq_body, n, s_blk, radii2, ks),
        grid=(grid,),
        in_specs=[
            pl.BlockSpec((s_blk, n), lambda i: (i, 0)),
            pl.BlockSpec((cw, cw), lambda i: (0, 0)),
        ],
        out_specs=[pl.BlockSpec((s_blk, k), lambda i: (i, 0)) for k in ks],
        out_shape=[jax.ShapeDtypeStruct((s, k), jnp.int32) for k in ks],
        scratch_shapes=[pltpu.VMEM((s_blk, n), jnp.float32)],
    )(sqr, tri)
    return list(outs)


def _sa_msg(xyz, points, npoint, radius_list, nsample_list, scale_params):
    fps_idx = _fps(xyz, npoint)
    new_xyz = xyz[fps_idx]
    gis = _ball_query_tc(radius_list, nsample_list, xyz, new_xyz)
    outs = []
    for i, radius in enumerate(radius_list):
        k = nsample_list[i]
        gi = gis[i]
        grouped_xyz = xyz[gi] - new_xyz[:, None, :]
        grouped = jnp.concatenate([points[gi], grouped_xyz], axis=-1)
        s, _, c = grouped.shape
        feat = grouped.reshape(s * k, c)
        layers = _fold_layers(scale_params[i])
        outs.append(_mlp_pallas(feat, layers, pool_k=k))
    return new_xyz, jnp.concatenate(outs, axis=-1)


def _fp(xyz1, xyz2, points1, points2, layers):
    n = xyz1.shape[0]
    s = xyz2.shape[0]
    if s == 1:
        interp = jnp.broadcast_to(points2, (n, points2.shape[-1]))
    else:
        dists = (
            jnp.sum(xyz1 ** 2, -1)[:, None]
            + jnp.sum(xyz2 ** 2, -1)[None, :]
            - 2.0 * jnp.dot(xyz1, xyz2.T)
        )
        neg, idx = jax.lax.top_k(-dists, 3)
        d = -neg
        recip = 1.0 / (d + 1e-8)
        w = recip / jnp.sum(recip, -1, keepdims=True)
        interp = jnp.sum(points2[idx] * w[..., None], axis=1)
    x = jnp.concatenate([points1, interp], axis=-1)
    return _mlp_pallas(x, _fold_layers(layers))


def _head(x, p):
    layers = _fold_layers([p['l1']], last_plain=p['l2'])
    return _mlp_pallas(x, layers)


def _bin_vals():
    bb = np.array([0, 0.00794435329, 0.0158887021, 0.0238330509, 0.0317773996,
                   0.0397217484, 0.0476660972, 0.055610446, 0.0635547948,
                   0.0714991435, 0.08], dtype=np.float32)
    bv = (bb[:-1] + bb[1:]) / 2.0
    bv = np.minimum(bv, np.float32(0.08 - 0.005))
    return jnp.asarray(bv)


def _normalize(x):
    nrm = jnp.sqrt(jnp.sum(x * x, axis=-1, keepdims=True))
    return x / jnp.maximum(nrm, 1e-12)


def kernel(xyz_pc, params):
    xyz0 = xyz_pc[0]
    p = params

    l1_xyz, l1_points = _sa_msg(xyz0, xyz0, 2048, [0.02, 0.04, 0.08],
                                [32, 64, 128], p['sa1'])
    l2_xyz, l2_points = _sa_msg(l1_xyz, l1_points, 512, [0.04, 0.08, 0.16],
                                [64, 64, 128], p['sa2'])
    l3_xyz, l3_points = _sa_msg(l2_xyz, l2_points, 128, [0.08, 0.16, 0.32],
                                [64, 64, 128], p['sa3'])

    x4 = jnp.concatenate([l3_xyz, l3_points], axis=-1)
    l4_points = _mlp_pallas(x4, _fold_layers(p['sa4']), pool_k=x4.shape[0])
    l4_xyz = jnp.zeros((1, 3), xyz0.dtype)

    l3_points = _fp(l3_xyz, l4_xyz, l3_points, l4_points, p['fp3'])
    l2_points = _fp(l2_xyz, l3_xyz, l2_points, l3_points, p['fp2'])
    l1_points = _fp(l1_xyz, l2_xyz, l1_points, l2_points, p['fp1'])

    pred_points = l1_xyz
    grasp_dir = _normalize(_head(l1_points, p['dir']))
    approach = _head(l1_points, p['app'])
    dot = jnp.sum(approach * grasp_dir, axis=-1, keepdims=True)
    approach = _normalize(approach - dot * grasp_dir)
    width_logits = _head(l1_points, p['off'])
    seg = _head(l1_points, p['seg'])

    bin_vals = _bin_vals()
    width_idx = jnp.argmax(width_logits, axis=-1)
    grasp_width = bin_vals[width_idx][..., None]
    gripper_depth = 0.1034
    grasp_r = jnp.stack([grasp_dir, jnp.cross(approach, grasp_dir), approach],
                        axis=2)
    grasp_t = (pred_points + grasp_width / 2.0 * grasp_dir
               - gripper_depth * approach)[..., None]
    np_ = approach.shape[0]
    homog = jnp.concatenate([jnp.zeros((np_, 1, 3), jnp.float32),
                             jnp.ones((np_, 1, 1), jnp.float32)], axis=2)
    pred_grasps = jnp.concatenate(
        [jnp.concatenate([grasp_r, grasp_t], axis=2), homog], axis=1)

    pred_scores = jax.nn.sigmoid(seg)
    return (pred_grasps[None], pred_scores[None], pred_points[None])
